# baseline (speedup 1.0000x reference)
import functools

import jax
import jax.numpy as jnp
from jax import lax
from jax.experimental import pallas as pl
from jax.experimental.pallas import tpu as pltpu
from jax.experimental.pallas import tpu_sc as plsc

N = 10000
E = 320000
D = 128
R = 32

NC = 2
NS = 16
NW = NC * NS
CH = 80
NP = 10240
ZR = NP // NS
ZB = 160

BE = 2000


def _pipe(nch, fire, process, drain):
    fire(0, 0)
    fire(1, 1)
    if nch % 2 == 0:
        @pl.loop(0, nch - 2, step=2)
        def _(k):
            process(k, 0)
            fire(k + 2, 0)
            process(k + 1, 1)
            fire(k + 3, 1)

        process(nch - 2, 0)
        process(nch - 1, 1)
    else:
        @pl.loop(0, nch - 2, step=2)
        def _(k):
            process(k, 0)
            fire(k + 2, 0)
            process(k + 1, 1)
            fire(jnp.minimum(k + 3, nch - 1), 1)

        process(nch - 1, 0)
        drain(nch - 1, 1)


def _sc_mesh():
    return plsc.VectorSubcoreMesh(
        core_axis_name="c", subcore_axis_name="s",
        num_cores=NC, num_subcores=NS)


def _node_pre_body(nf_ref, sgA_ref, sgB_ref, sgb_ref, dgA_ref, dgB_ref,
                   dgb_ref, duA_ref, duB_ref, dub_ref,
                   suA_ref, suB_ref, sub_ref,
                   es_ref, ed_ref, bh_ref, su_ref):
    nf = nf_ref[...]
    f32 = jnp.float32

    def lr(A_ref, B_ref, b_ref):
        return jnp.dot(jnp.dot(nf, A_ref[...], preferred_element_type=f32),
                       B_ref[...], preferred_element_type=f32) + b_ref[...]

    es_ref[...] = lr(sgA_ref, sgB_ref, sgb_ref)
    ed_ref[...] = lr(dgA_ref, dgB_ref, dgb_ref)
    bh_ref[...] = lr(duA_ref, duB_ref, dub_ref)
    su_ref[...] = lr(suA_ref, suB_ref, sub_ref)


def _node_pre(nf, sgA, sgB, sgb, dgA, dgB, dgb, duA, duB, dub,
              suA, suB, sub):
    f32 = jnp.float32
    return pl.pallas_call(
        _node_pre_body,
        out_shape=(
            jax.ShapeDtypeStruct((N, D), f32),
            jax.ShapeDtypeStruct((N, D), f32),
            jax.ShapeDtypeStruct((N, D), f32),
            jax.ShapeDtypeStruct((N, D), f32),
        ),
    )(nf, sgA, sgB, sgb, dgA, dgB, dgb, duA, duB, dub, suA, suB, sub)


def _gather_g(es, ed, src, dst):
    f32 = jnp.float32
    ne = src.shape[0]
    per_tile = ne // NW
    nch = per_tile // CH

    @functools.partial(
        pl.kernel,
        out_type=jax.ShapeDtypeStruct((ne, D // 2), f32),
        mesh=_sc_mesh(),
        compiler_params=pltpu.CompilerParams(needs_layout_passes=False),
        scratch_types=[
            pltpu.VMEM((per_tile,), jnp.int32),
            pltpu.VMEM((per_tile,), jnp.int32),
            pltpu.VMEM((CH, D), f32),
            pltpu.VMEM((CH, D), f32),
            pltpu.VMEM((CH, D), f32),
            pltpu.VMEM((CH, D), f32),
            pltpu.VMEM((CH, D // 2), f32),
            pltpu.VMEM((CH, D // 2), f32),
            pltpu.SemaphoreType.DMA,
            pltpu.SemaphoreType.DMA,
        ],
    )
    def k(es_hbm, ed_hbm, src_hbm, dst_hbm, g_hbm,
          siv, div, gs0, gd0, gs1, gd1, go0, go1, gsem0, gsem1):
        wid = lax.axis_index("c") * NS + lax.axis_index("s")
        base = wid * per_tile
        pltpu.sync_copy(src_hbm.at[pl.ds(base, per_tile)], siv)
        pltpu.sync_copy(dst_hbm.at[pl.ds(base, per_tile)], div)

        bufs = ((gs0, gd0, go0, gsem0), (gs1, gd1, go1, gsem1))

        def mk(k, bi):
            gs, gd, _, sem = bufs[bi]
            off = k * CH
            return (pltpu.make_async_copy(es_hbm.at[siv.at[pl.ds(off, CH)]],
                                          gs, sem),
                    pltpu.make_async_copy(ed_hbm.at[div.at[pl.ds(off, CH)]],
                                          gd, sem))

        def fire(k, bi):
            for cp in mk(k, bi):
                cp.start()

        def drain(k, bi):
            for cp in mk(k, bi):
                cp.wait()

        def process(k, bi):
            gs, gd, go, _ = bufs[bi]
            drain(k, bi)

            @pl.loop(0, CH)
            def _(r):
                for gi in range(D // 32):
                    a = gs[r, pl.ds(gi * 32, 16)] + gd[r, pl.ds(gi * 32, 16)]
                    b = (gs[r, pl.ds(gi * 32 + 16, 16)]
                         + gd[r, pl.ds(gi * 32 + 16, 16)])
                    pk = plsc.pack(a, b, format=plsc.PackFormat.INTERLEAVED)
                    go[r, pl.ds(gi * 16, 16)] = plsc.bitcast(pk, jnp.float32)

            pltpu.sync_copy(go, g_hbm.at[pl.ds(base + k * CH, CH)])

        _pipe(nch, fire, process, drain)

    return k(es, ed, src, dst)


def _edge_body(ef_ref, g_ref, egA_ref, egB_ref, egb_ref, lng_ref, lnb_ref,
               *refs, has_ybuf):
    y_ref, sig_ref = refs[-2:]
    f32 = jnp.float32
    ef = ef_ref[...]
    nbe = g_ref.shape[0]
    gw = lax.bitcast_convert_type(g_ref[...], jnp.uint32)
    a = lax.bitcast_convert_type(gw << 16, f32)
    b = lax.bitcast_convert_type(gw & jnp.uint32(0xFFFF0000), f32)
    gn = jnp.concatenate([a.reshape(nbe, 4, 1, 16),
                          b.reshape(nbe, 4, 1, 16)], axis=2).reshape(nbe, D)
    m = (gn
         + jnp.dot(jnp.dot(ef, egA_ref[...], preferred_element_type=f32),
                   egB_ref[...], preferred_element_type=f32)
         + egb_ref[...])
    sig_ref[...] = jax.nn.sigmoid(m)
    mu = jnp.mean(m, axis=-1, keepdims=True)
    zc = m - mu
    var = jnp.mean(zc * zc, axis=-1, keepdims=True)
    yn = zc * lax.rsqrt(var + 1e-5) * lng_ref[...] + lnb_ref[...]
    y_ref[...] = ef + yn * jax.nn.sigmoid(yn)


def _edge_dense(ef, g, egA, egB, egb, lng, lnb, y_buf, blk_off):
    f32 = jnp.float32
    ne = g.shape[0]
    nblk = ne // BE
    in_specs = [
        pl.BlockSpec((BE, D), lambda i: (i + blk_off, 0)),
        pl.BlockSpec((BE, D // 2), lambda i: (i, 0)),
        pl.BlockSpec((D, R), lambda i: (0, 0)),
        pl.BlockSpec((R, D), lambda i: (0, 0)),
        pl.BlockSpec((1, D), lambda i: (0, 0)),
        pl.BlockSpec((1, D), lambda i: (0, 0)),
        pl.BlockSpec((1, D), lambda i: (0, 0)),
    ]
    args = [ef, g, egA, egB, egb, lng, lnb]
    aliases = {}
    if y_buf is not None:
        in_specs.append(pl.BlockSpec(memory_space=pl.ANY))
        args.append(y_buf)
        aliases = {7: 0}
    return pl.pallas_call(
        functools.partial(_edge_body, has_ybuf=y_buf is not None),
        grid=(nblk,),
        in_specs=in_specs,
        out_specs=(
            pl.BlockSpec((BE, D), lambda i: (i + blk_off, 0)),
            pl.BlockSpec((BE, D), lambda i: (i, 0)),
        ),
        out_shape=(
            jax.ShapeDtypeStruct((E, D), f32),
            jax.ShapeDtypeStruct((ne, D), f32),
        ),
        input_output_aliases=aliases,
    )(*args)


def _segment_sums(sigma, bh, src, dst):
    f32 = jnp.float32
    CHD = 40
    ne = src.shape[0]
    per_tile = ne // NW

    nch = per_tile // CHD

    @functools.partial(
        pl.kernel,
        out_type=(
            jax.ShapeDtypeStruct((NC, NP, D), f32),
            jax.ShapeDtypeStruct((NC, NP, D), f32),
        ),
        mesh=_sc_mesh(),
        scratch_types=[
            pltpu.VMEM_SHARED((NP, D), f32),
            pltpu.VMEM((per_tile,), jnp.int32),
            pltpu.VMEM((CHD,), jnp.int32),
            pltpu.VMEM((CHD,), jnp.int32),
            pltpu.VMEM((CHD, D), f32),
            pltpu.VMEM((CHD, D), f32),
            pltpu.VMEM((CHD, D), f32),
            pltpu.VMEM((CHD, D), f32),
            pltpu.SemaphoreType.DMA,
            pltpu.SemaphoreType.DMA,
        ],
    )
    def k(sig_hbm, bh_hbm, src_hbm, dst_hbm, ss_hbm, ssh_hbm,
          acc, siv, dv0, dv1, sg0, sg1, bh0, bh1, sem0, sem1):
        c = lax.axis_index("c")
        s = lax.axis_index("s")
        base = (c * NS + s) * per_tile
        zvec = jnp.zeros((16,), f32)

        pltpu.sync_copy(src_hbm.at[pl.ds(base, per_tile)], siv)

        @pl.loop(0, CHD)
        def _(r):
            for g in range(D // 16):
                bh0[r, pl.ds(g * 16, 16)] = zvec

        def zero_acc():
            for kk in range(ZR // CHD):
                pltpu.sync_copy(bh0, acc.at[pl.ds(s * ZR + kk * CHD, CHD)])

        bufs = ((dv0, sg0, bh0, sem0), (dv1, sg1, bh1, sem1))

        zero_acc()
        plsc.subcore_barrier()

        def mk1(k, bi):
            dv, sg, _, sem = bufs[bi]
            off = k * CHD
            return (pltpu.make_async_copy(sig_hbm.at[pl.ds(base + off, CHD)],
                                          sg, sem),
                    pltpu.make_async_copy(dst_hbm.at[pl.ds(base + off, CHD)],
                                          dv, sem))

        def fire1(k, bi):
            for cp in mk1(k, bi):
                cp.start()

        def drain1(k, bi):
            for cp in mk1(k, bi):
                cp.wait()

        def process1(k, bi):
            dv, sg, _, _ = bufs[bi]
            drain1(k, bi)
            pltpu.sync_copy(sg, acc.at[dv], add=True)

        _pipe(nch, fire1, process1, drain1)

        plsc.subcore_barrier()
        pltpu.sync_copy(acc.at[pl.ds(s * ZR, ZR)],
                        ss_hbm.at[c].at[pl.ds(s * ZR, ZR)])
        plsc.subcore_barrier()

        zero_acc()
        plsc.subcore_barrier()

        def mk2(k, bi):
            dv, sg, bh, sem = bufs[bi]
            off = k * CHD
            return (pltpu.make_async_copy(sig_hbm.at[pl.ds(base + off, CHD)],
                                          sg, sem),
                    pltpu.make_async_copy(dst_hbm.at[pl.ds(base + off, CHD)],
                                          dv, sem),
                    pltpu.make_async_copy(bh_hbm.at[siv.at[pl.ds(off, CHD)]],
                                          bh, sem))

        def fire2(k, bi):
            for cp in mk2(k, bi):
                cp.start()

        def drain2(k, bi):
            for cp in mk2(k, bi):
                cp.wait()

        def process2(k, bi):
            dv, sg, bh, _ = bufs[bi]
            drain2(k, bi)

            @pl.loop(0, CHD)
            def _(r):
                for g in range(D // 16):
                    sl = pl.ds(g * 16, 16)
                    bh[r, sl] = bh[r, sl] * sg[r, sl]

            pltpu.sync_copy(bh, acc.at[dv], add=True)

        _pipe(nch, fire2, process2, drain2)

        plsc.subcore_barrier()
        pltpu.sync_copy(acc.at[pl.ds(s * ZR, ZR)],
                        ssh_hbm.at[c].at[pl.ds(s * ZR, ZR)])

    return k(sigma, bh, src, dst)


def _node_post_body(nf_ref, su_ref, ss1_ref, ssh1_ref, ss2_ref, ssh2_ref,
                    ss3_ref, ssh3_ref, lng_ref, lnb_ref, x_ref):
    ss = (ss1_ref[0] + ss1_ref[1] + ss2_ref[0] + ss2_ref[1]
          + ss3_ref[0] + ss3_ref[1])
    ssh = (ssh1_ref[0] + ssh1_ref[1] + ssh2_ref[0] + ssh2_ref[1]
           + ssh3_ref[0] + ssh3_ref[1])
    h = ssh / (ss + 1e-6)
    t = su_ref[...] + h
    mu = jnp.mean(t, axis=-1, keepdims=True)
    zc = t - mu
    var = jnp.mean(zc * zc, axis=-1, keepdims=True)
    tn = zc * lax.rsqrt(var + 1e-5) * lng_ref[...] + lnb_ref[...]
    x_ref[...] = nf_ref[...] + tn * jax.nn.sigmoid(tn)


def _node_post(nf, su, parts, lng, lnb):
    BN = 2000
    nspec = pl.BlockSpec((BN, D), lambda i: (i, 0))
    pspec = pl.BlockSpec((2, BN, D), lambda i: (0, i, 0))
    wspec = pl.BlockSpec((1, D), lambda i: (0, 0))
    flat = [a for pair in parts for a in pair]
    return pl.pallas_call(
        _node_post_body,
        grid=(N // BN,),
        in_specs=[nspec, nspec] + [pspec] * 6 + [wspec, wspec],
        out_specs=nspec,
        out_shape=jax.ShapeDtypeStruct((N, D), jnp.float32),
    )(nf, su, *flat, lng, lnb)


def kernel(node_feats, edge_feats, edge_index,
           sg_A, sg_B, sg_b, dg_A, dg_B, dg_b, eg_A, eg_B, eg_b,
           du_A, du_B, du_b, su_A, su_B, su_b,
           ln_n_g, ln_n_b, ln_e_g, ln_e_b):
    src = edge_index[0].astype(jnp.int32)
    dst = edge_index[1].astype(jnp.int32)

    r1 = lambda v: v.reshape(1, D)

    es, ed, bh, su = _node_pre(node_feats, sg_A, sg_B, r1(sg_b),
                               dg_A, dg_B, r1(dg_b), du_A, du_B, r1(du_b),
                               su_A, su_B, r1(su_b))

    bounds = [0, 64000, 192000, E]
    srcs = [src[bounds[i]:bounds[i + 1]] for i in range(3)]
    dsts = [dst[bounds[i]:bounds[i + 1]] for i in range(3)]

    gs = [_gather_g(es, ed, srcs[i], dsts[i]) for i in range(3)]

    y = None
    sigs = []
    for i in range(3):
        y, sig = _edge_dense(edge_feats, gs[i], eg_A, eg_B, r1(eg_b),
                             r1(ln_e_g), r1(ln_e_b), y, bounds[i] // BE)
        sigs.append(sig)

    parts = [_segment_sums(sigs[i], bh, srcs[i], dsts[i]) for i in range(3)]

    x = _node_post(node_feats, su, parts, r1(ln_n_g), r1(ln_n_b))
    return (x, y)


# speedup vs baseline: 2.2110x; 2.2110x over previous
import functools

import jax
import jax.numpy as jnp
from jax import lax
from jax.experimental import pallas as pl
from jax.experimental.pallas import tpu as pltpu
from jax.experimental.pallas import tpu_sc as plsc

N = 10000
E = 320000
D = 128
R = 32

NC = 2
NS = 16
NW = NC * NS
CH = 80
NP = 10240
ZR = NP // NS
ZB = 160

BE = 2000


def _pipe(nch, fire, process, drain):
    fire(0, 0)
    fire(1, 1)
    if nch % 2 == 0:
        @pl.loop(0, nch - 2, step=2)
        def _(k):
            process(k, 0)
            fire(k + 2, 0)
            process(k + 1, 1)
            fire(k + 3, 1)

        process(nch - 2, 0)
        process(nch - 1, 1)
    else:
        @pl.loop(0, nch - 2, step=2)
        def _(k):
            process(k, 0)
            fire(k + 2, 0)
            process(k + 1, 1)
            fire(jnp.minimum(k + 3, nch - 1), 1)

        process(nch - 1, 0)
        drain(nch - 1, 1)


def _sc_mesh():
    return plsc.VectorSubcoreMesh(
        core_axis_name="c", subcore_axis_name="s",
        num_cores=NC, num_subcores=NS)


def _node_pre_body(nf_ref, sgA_ref, sgB_ref, sgb_ref, dgA_ref, dgB_ref,
                   dgb_ref, duA_ref, duB_ref, dub_ref,
                   suA_ref, suB_ref, sub_ref,
                   es_ref, ed_ref, bh_ref, su_ref):
    nf = nf_ref[...]
    f32 = jnp.float32

    def lr(A_ref, B_ref, b_ref):
        return jnp.dot(jnp.dot(nf, A_ref[...], preferred_element_type=f32),
                       B_ref[...], preferred_element_type=f32) + b_ref[...]

    es_ref[...] = lr(sgA_ref, sgB_ref, sgb_ref)
    ed_ref[...] = lr(dgA_ref, dgB_ref, dgb_ref)
    bh_ref[...] = lr(duA_ref, duB_ref, dub_ref)
    su_ref[...] = lr(suA_ref, suB_ref, sub_ref)


def _node_pre(nf, sgA, sgB, sgb, dgA, dgB, dgb, duA, duB, dub,
              suA, suB, sub):
    f32 = jnp.float32
    return pl.pallas_call(
        _node_pre_body,
        out_shape=(
            jax.ShapeDtypeStruct((N, D), f32),
            jax.ShapeDtypeStruct((N, D), f32),
            jax.ShapeDtypeStruct((N, D), f32),
            jax.ShapeDtypeStruct((N, D), f32),
        ),
    )(nf, sgA, sgB, sgb, dgA, dgB, dgb, duA, duB, dub, suA, suB, sub)


def _gather_g(es, ed, src, dst):
    f32 = jnp.float32
    ne = src.shape[0]
    per_tile = ne // NW
    nch = per_tile // CH

    @functools.partial(
        pl.kernel,
        out_type=jax.ShapeDtypeStruct((ne, D), f32),
        mesh=_sc_mesh(),
        scratch_types=[
            pltpu.VMEM((per_tile,), jnp.int32),
            pltpu.VMEM((per_tile,), jnp.int32),
            pltpu.VMEM((CH, D), f32),
            pltpu.VMEM((CH, D), f32),
            pltpu.VMEM((CH, D), f32),
            pltpu.VMEM((CH, D), f32),
            pltpu.SemaphoreType.DMA,
            pltpu.SemaphoreType.DMA,
        ],
    )
    def k(es_hbm, ed_hbm, src_hbm, dst_hbm, g_hbm,
          siv, div, gs0, gd0, gs1, gd1, gsem0, gsem1):
        wid = lax.axis_index("c") * NS + lax.axis_index("s")
        base = wid * per_tile
        pltpu.sync_copy(src_hbm.at[pl.ds(base, per_tile)], siv)
        pltpu.sync_copy(dst_hbm.at[pl.ds(base, per_tile)], div)

        bufs = ((gs0, gd0, gsem0), (gs1, gd1, gsem1))

        def mk(k, bi):
            gs, gd, sem = bufs[bi]
            off = k * CH
            return (pltpu.make_async_copy(es_hbm.at[siv.at[pl.ds(off, CH)]],
                                          gs, sem),
                    pltpu.make_async_copy(ed_hbm.at[div.at[pl.ds(off, CH)]],
                                          gd, sem))

        def fire(k, bi):
            for cp in mk(k, bi):
                cp.start()

        def drain(k, bi):
            for cp in mk(k, bi):
                cp.wait()

        def process(k, bi):
            gs, gd, _ = bufs[bi]
            drain(k, bi)

            @pl.loop(0, CH)
            def _(r):
                for gi in range(D // 16):
                    sl = pl.ds(gi * 16, 16)
                    gs[r, sl] = gs[r, sl] + gd[r, sl]

            pltpu.sync_copy(gs, g_hbm.at[pl.ds(base + k * CH, CH)])

        _pipe(nch, fire, process, drain)

    return k(es, ed, src, dst)


def _edge_body(ef_ref, g_ref, egA_ref, egB_ref, egb_ref, lng_ref, lnb_ref,
               *refs, has_ybuf):
    y_ref, sig_ref = refs[-2:]
    f32 = jnp.float32
    ef = ef_ref[...]
    m = (g_ref[...]
         + jnp.dot(jnp.dot(ef, egA_ref[...], preferred_element_type=f32),
                   egB_ref[...], preferred_element_type=f32)
         + egb_ref[...])
    sig_ref[...] = jax.nn.sigmoid(m)
    mu = jnp.mean(m, axis=-1, keepdims=True)
    zc = m - mu
    var = jnp.mean(zc * zc, axis=-1, keepdims=True)
    yn = zc * lax.rsqrt(var + 1e-5) * lng_ref[...] + lnb_ref[...]
    y_ref[...] = ef + yn * jax.nn.sigmoid(yn)


def _edge_dense(ef, g, egA, egB, egb, lng, lnb, y_buf, blk_off):
    f32 = jnp.float32
    ne = g.shape[0]
    nblk = ne // BE
    in_specs = [
        pl.BlockSpec((BE, D), lambda i: (i + blk_off, 0)),
        pl.BlockSpec((BE, D), lambda i: (i, 0)),
        pl.BlockSpec((D, R), lambda i: (0, 0)),
        pl.BlockSpec((R, D), lambda i: (0, 0)),
        pl.BlockSpec((1, D), lambda i: (0, 0)),
        pl.BlockSpec((1, D), lambda i: (0, 0)),
        pl.BlockSpec((1, D), lambda i: (0, 0)),
    ]
    args = [ef, g, egA, egB, egb, lng, lnb]
    aliases = {}
    if y_buf is not None:
        in_specs.append(pl.BlockSpec(memory_space=pl.ANY))
        args.append(y_buf)
        aliases = {7: 0}
    return pl.pallas_call(
        functools.partial(_edge_body, has_ybuf=y_buf is not None),
        grid=(nblk,),
        in_specs=in_specs,
        out_specs=(
            pl.BlockSpec((BE, D), lambda i: (i + blk_off, 0)),
            pl.BlockSpec((BE, D), lambda i: (i, 0)),
        ),
        out_shape=(
            jax.ShapeDtypeStruct((E, D), f32),
            jax.ShapeDtypeStruct((ne, D), f32),
        ),
        input_output_aliases=aliases,
    )(*args)


def _segment_sums(sigma, bh, src, dst):
    f32 = jnp.float32
    CHD = 40
    ne = src.shape[0]
    per_tile = ne // NW

    nch = per_tile // CHD

    @functools.partial(
        pl.kernel,
        out_type=(
            jax.ShapeDtypeStruct((NC, NP, D), f32),
            jax.ShapeDtypeStruct((NC, NP, D), f32),
        ),
        mesh=_sc_mesh(),
        scratch_types=[
            pltpu.VMEM_SHARED((NP, D), f32),
            pltpu.VMEM((per_tile,), jnp.int32),
            pltpu.VMEM((CHD,), jnp.int32),
            pltpu.VMEM((CHD,), jnp.int32),
            pltpu.VMEM((CHD, D), f32),
            pltpu.VMEM((CHD, D), f32),
            pltpu.VMEM((CHD, D), f32),
            pltpu.VMEM((CHD, D), f32),
            pltpu.SemaphoreType.DMA,
            pltpu.SemaphoreType.DMA,
        ],
    )
    def k(sig_hbm, bh_hbm, src_hbm, dst_hbm, ss_hbm, ssh_hbm,
          acc, siv, dv0, dv1, sg0, sg1, bh0, bh1, sem0, sem1):
        c = lax.axis_index("c")
        s = lax.axis_index("s")
        base = (c * NS + s) * per_tile
        zvec = jnp.zeros((16,), f32)

        pltpu.sync_copy(src_hbm.at[pl.ds(base, per_tile)], siv)

        @pl.loop(0, CHD)
        def _(r):
            for g in range(D // 16):
                bh0[r, pl.ds(g * 16, 16)] = zvec

        def zero_acc():
            for kk in range(ZR // CHD):
                pltpu.sync_copy(bh0, acc.at[pl.ds(s * ZR + kk * CHD, CHD)])

        bufs = ((dv0, sg0, bh0, sem0), (dv1, sg1, bh1, sem1))

        zero_acc()
        plsc.subcore_barrier()

        def mk1(k, bi):
            dv, sg, _, sem = bufs[bi]
            off = k * CHD
            return (pltpu.make_async_copy(sig_hbm.at[pl.ds(base + off, CHD)],
                                          sg, sem),
                    pltpu.make_async_copy(dst_hbm.at[pl.ds(base + off, CHD)],
                                          dv, sem))

        def fire1(k, bi):
            for cp in mk1(k, bi):
                cp.start()

        def drain1(k, bi):
            for cp in mk1(k, bi):
                cp.wait()

        def process1(k, bi):
            dv, sg, _, _ = bufs[bi]
            drain1(k, bi)
            pltpu.sync_copy(sg, acc.at[dv], add=True)

        _pipe(nch, fire1, process1, drain1)

        plsc.subcore_barrier()
        pltpu.sync_copy(acc.at[pl.ds(s * ZR, ZR)],
                        ss_hbm.at[c].at[pl.ds(s * ZR, ZR)])
        plsc.subcore_barrier()

        zero_acc()
        plsc.subcore_barrier()

        def mk2(k, bi):
            dv, sg, bh, sem = bufs[bi]
            off = k * CHD
            return (pltpu.make_async_copy(sig_hbm.at[pl.ds(base + off, CHD)],
                                          sg, sem),
                    pltpu.make_async_copy(dst_hbm.at[pl.ds(base + off, CHD)],
                                          dv, sem),
                    pltpu.make_async_copy(bh_hbm.at[siv.at[pl.ds(off, CHD)]],
                                          bh, sem))

        def fire2(k, bi):
            for cp in mk2(k, bi):
                cp.start()

        def drain2(k, bi):
            for cp in mk2(k, bi):
                cp.wait()

        def process2(k, bi):
            dv, sg, bh, _ = bufs[bi]
            drain2(k, bi)

            @pl.loop(0, CHD)
            def _(r):
                for g in range(D // 16):
                    sl = pl.ds(g * 16, 16)
                    bh[r, sl] = bh[r, sl] * sg[r, sl]

            pltpu.sync_copy(bh, acc.at[dv], add=True)

        _pipe(nch, fire2, process2, drain2)

        plsc.subcore_barrier()
        pltpu.sync_copy(acc.at[pl.ds(s * ZR, ZR)],
                        ssh_hbm.at[c].at[pl.ds(s * ZR, ZR)])

    return k(sigma, bh, src, dst)


def _node_post_body(nf_ref, su_ref, ss1_ref, ssh1_ref, ss2_ref, ssh2_ref,
                    ss3_ref, ssh3_ref, lng_ref, lnb_ref, x_ref):
    ss = (ss1_ref[0] + ss1_ref[1] + ss2_ref[0] + ss2_ref[1]
          + ss3_ref[0] + ss3_ref[1])
    ssh = (ssh1_ref[0] + ssh1_ref[1] + ssh2_ref[0] + ssh2_ref[1]
           + ssh3_ref[0] + ssh3_ref[1])
    h = ssh / (ss + 1e-6)
    t = su_ref[...] + h
    mu = jnp.mean(t, axis=-1, keepdims=True)
    zc = t - mu
    var = jnp.mean(zc * zc, axis=-1, keepdims=True)
    tn = zc * lax.rsqrt(var + 1e-5) * lng_ref[...] + lnb_ref[...]
    x_ref[...] = nf_ref[...] + tn * jax.nn.sigmoid(tn)


def _node_post(nf, su, parts, lng, lnb):
    BN = 2000
    nspec = pl.BlockSpec((BN, D), lambda i: (i, 0))
    pspec = pl.BlockSpec((2, BN, D), lambda i: (0, i, 0))
    wspec = pl.BlockSpec((1, D), lambda i: (0, 0))
    flat = [a for pair in parts for a in pair]
    return pl.pallas_call(
        _node_post_body,
        grid=(N // BN,),
        in_specs=[nspec, nspec] + [pspec] * 6 + [wspec, wspec],
        out_specs=nspec,
        out_shape=jax.ShapeDtypeStruct((N, D), jnp.float32),
    )(nf, su, *flat, lng, lnb)


def kernel(node_feats, edge_feats, edge_index,
           sg_A, sg_B, sg_b, dg_A, dg_B, dg_b, eg_A, eg_B, eg_b,
           du_A, du_B, du_b, su_A, su_B, su_b,
           ln_n_g, ln_n_b, ln_e_g, ln_e_b):
    src = edge_index[0].astype(jnp.int32)
    dst = edge_index[1].astype(jnp.int32)

    r1 = lambda v: v.reshape(1, D)

    es, ed, bh, su = _node_pre(node_feats, sg_A, sg_B, r1(sg_b),
                               dg_A, dg_B, r1(dg_b), du_A, du_B, r1(du_b),
                               su_A, su_B, r1(su_b))

    bounds = [0, 64000, 192000, E]
    srcs = [src[bounds[i]:bounds[i + 1]] for i in range(3)]
    dsts = [dst[bounds[i]:bounds[i + 1]] for i in range(3)]

    gs = [_gather_g(es, ed, srcs[i], dsts[i]) for i in range(3)]

    y = None
    sigs = []
    for i in range(3):
        y, sig = _edge_dense(edge_feats, gs[i], eg_A, eg_B, r1(eg_b),
                             r1(ln_e_g), r1(ln_e_b), y, bounds[i] // BE)
        sigs.append(sig)

    parts = [_segment_sums(sigs[i], bh, srcs[i], dsts[i]) for i in range(3)]

    x = _node_post(node_feats, su, parts, r1(ln_n_g), r1(ln_n_b))
    return (x, y)


# speedup vs baseline: 2.2957x; 1.0383x over previous
import functools

import jax
import jax.numpy as jnp
from jax import lax
from jax.experimental import pallas as pl
from jax.experimental.pallas import tpu as pltpu
from jax.experimental.pallas import tpu_sc as plsc

N = 10000
E = 320000
D = 128
R = 32

NC = 2
NS = 16
NW = NC * NS
CH = 80
NP = 10240
ZR = NP // NS
ZB = 160

BE = 2000


def _pipe(nch, fire, process, drain):
    fire(0, 0)
    fire(1, 1)
    if nch % 2 == 0:
        @pl.loop(0, nch - 2, step=2)
        def _(k):
            process(k, 0)
            fire(k + 2, 0)
            process(k + 1, 1)
            fire(k + 3, 1)

        process(nch - 2, 0)
        process(nch - 1, 1)
    else:
        @pl.loop(0, nch - 2, step=2)
        def _(k):
            process(k, 0)
            fire(k + 2, 0)
            process(k + 1, 1)
            fire(jnp.minimum(k + 3, nch - 1), 1)

        process(nch - 1, 0)
        drain(nch - 1, 1)


def _sc_mesh():
    return plsc.VectorSubcoreMesh(
        core_axis_name="c", subcore_axis_name="s",
        num_cores=NC, num_subcores=NS)


def _node_pre_body(nf_ref, sgA_ref, sgB_ref, sgb_ref, dgA_ref, dgB_ref,
                   dgb_ref, duA_ref, duB_ref, dub_ref,
                   suA_ref, suB_ref, sub_ref,
                   es_ref, ed_ref, bh_ref, su_ref):
    nf = nf_ref[...]
    f32 = jnp.float32

    def lr(A_ref, B_ref, b_ref):
        return jnp.dot(jnp.dot(nf, A_ref[...], preferred_element_type=f32),
                       B_ref[...], preferred_element_type=f32) + b_ref[...]

    es_ref[...] = lr(sgA_ref, sgB_ref, sgb_ref)
    ed_ref[...] = lr(dgA_ref, dgB_ref, dgb_ref)
    bh_ref[...] = lr(duA_ref, duB_ref, dub_ref)
    su_ref[...] = lr(suA_ref, suB_ref, sub_ref)


def _node_pre(nf, sgA, sgB, sgb, dgA, dgB, dgb, duA, duB, dub,
              suA, suB, sub):
    f32 = jnp.float32
    return pl.pallas_call(
        _node_pre_body,
        out_shape=(
            jax.ShapeDtypeStruct((N, D), f32),
            jax.ShapeDtypeStruct((N, D), f32),
            jax.ShapeDtypeStruct((N, D), f32),
            jax.ShapeDtypeStruct((N, D), f32),
        ),
    )(nf, sgA, sgB, sgb, dgA, dgB, dgb, duA, duB, dub, suA, suB, sub)


def _gather_g(es, ed, src, dst):
    f32 = jnp.float32
    ne = src.shape[0]
    per_tile = ne // NW
    nch = per_tile // CH

    @functools.partial(
        pl.kernel,
        out_type=jax.ShapeDtypeStruct((ne, D), f32),
        mesh=_sc_mesh(),
        scratch_types=[
            pltpu.VMEM((per_tile,), jnp.int32),
            pltpu.VMEM((per_tile,), jnp.int32),
            pltpu.VMEM((CH, D), f32),
            pltpu.VMEM((CH, D), f32),
            pltpu.VMEM((CH, D), f32),
            pltpu.VMEM((CH, D), f32),
            pltpu.SemaphoreType.DMA,
            pltpu.SemaphoreType.DMA,
        ],
    )
    def k(es_hbm, ed_hbm, src_hbm, dst_hbm, g_hbm,
          siv, div, gs0, gd0, gs1, gd1, gsem0, gsem1):
        wid = lax.axis_index("c") * NS + lax.axis_index("s")
        base = wid * per_tile
        pltpu.sync_copy(src_hbm.at[pl.ds(base, per_tile)], siv)
        pltpu.sync_copy(dst_hbm.at[pl.ds(base, per_tile)], div)

        bufs = ((gs0, gd0, gsem0), (gs1, gd1, gsem1))

        def mk(k, bi):
            gs, gd, sem = bufs[bi]
            off = k * CH
            return (pltpu.make_async_copy(es_hbm.at[siv.at[pl.ds(off, CH)]],
                                          gs, sem),
                    pltpu.make_async_copy(ed_hbm.at[div.at[pl.ds(off, CH)]],
                                          gd, sem))

        def fire(k, bi):
            for cp in mk(k, bi):
                cp.start()

        def drain(k, bi):
            for cp in mk(k, bi):
                cp.wait()

        def process(k, bi):
            gs, gd, _ = bufs[bi]
            drain(k, bi)

            @pl.loop(0, CH)
            def _(r):
                for gi in range(D // 16):
                    sl = pl.ds(gi * 16, 16)
                    gs[r, sl] = gs[r, sl] + gd[r, sl]

            pltpu.sync_copy(gs, g_hbm.at[pl.ds(base + k * CH, CH)])

        _pipe(nch, fire, process, drain)

    return k(es, ed, src, dst)


def _edge_body(ef_ref, g_ref, egA_ref, egB_ref, egb_ref, lng_ref, lnb_ref,
               *refs, has_ybuf):
    y_ref, sig_ref = refs[-2:]
    f32 = jnp.float32
    ef = ef_ref[...]
    m = (g_ref[...]
         + jnp.dot(jnp.dot(ef, egA_ref[...], preferred_element_type=f32),
                   egB_ref[...], preferred_element_type=f32)
         + egb_ref[...])
    sig_ref[...] = jax.nn.sigmoid(m)
    mu = jnp.mean(m, axis=-1, keepdims=True)
    zc = m - mu
    var = jnp.mean(zc * zc, axis=-1, keepdims=True)
    yn = zc * lax.rsqrt(var + 1e-5) * lng_ref[...] + lnb_ref[...]
    y_ref[...] = ef + yn * jax.nn.sigmoid(yn)


def _edge_dense(ef, g, egA, egB, egb, lng, lnb, y_buf, blk_off):
    f32 = jnp.float32
    ne = g.shape[0]
    nblk = ne // BE
    in_specs = [
        pl.BlockSpec((BE, D), lambda i: (i + blk_off, 0)),
        pl.BlockSpec((BE, D), lambda i: (i, 0)),
        pl.BlockSpec((D, R), lambda i: (0, 0)),
        pl.BlockSpec((R, D), lambda i: (0, 0)),
        pl.BlockSpec((1, D), lambda i: (0, 0)),
        pl.BlockSpec((1, D), lambda i: (0, 0)),
        pl.BlockSpec((1, D), lambda i: (0, 0)),
    ]
    args = [ef, g, egA, egB, egb, lng, lnb]
    aliases = {}
    if y_buf is not None:
        in_specs.append(pl.BlockSpec(memory_space=pl.ANY))
        args.append(y_buf)
        aliases = {7: 0}
    return pl.pallas_call(
        functools.partial(_edge_body, has_ybuf=y_buf is not None),
        grid=(nblk,),
        in_specs=in_specs,
        out_specs=(
            pl.BlockSpec((BE, D), lambda i: (i + blk_off, 0)),
            pl.BlockSpec((BE, D), lambda i: (i, 0)),
        ),
        out_shape=(
            jax.ShapeDtypeStruct((E, D), f32),
            jax.ShapeDtypeStruct((ne, D), f32),
        ),
        input_output_aliases=aliases,
    )(*args)


def _seg_ss(sig, dst):
    f32 = jnp.float32
    CHD = 80
    ne = dst.shape[0]
    per_tile = ne // NW
    nch = per_tile // CHD

    @functools.partial(
        pl.kernel,
        out_type=jax.ShapeDtypeStruct((NC, NP, D), f32),
        mesh=_sc_mesh(),
        scratch_types=[
            pltpu.VMEM_SHARED((NP, D), f32),
            pltpu.VMEM((CHD,), jnp.int32),
            pltpu.VMEM((CHD,), jnp.int32),
            pltpu.VMEM((CHD, D), f32),
            pltpu.VMEM((CHD, D), f32),
            pltpu.SemaphoreType.DMA,
            pltpu.SemaphoreType.DMA,
        ],
    )
    def k(sig_hbm, dst_hbm, ss_hbm, acc, dv0, dv1, sg0, sg1, sem0, sem1):
        c = lax.axis_index("c")
        s = lax.axis_index("s")
        base = (c * NS + s) * per_tile
        zvec = jnp.zeros((16,), f32)

        @pl.loop(0, CHD)
        def _(r):
            for g in range(D // 16):
                sg0[r, pl.ds(g * 16, 16)] = zvec

        for kk in range(ZR // CHD):
            pltpu.sync_copy(sg0, acc.at[pl.ds(s * ZR + kk * CHD, CHD)])
        plsc.subcore_barrier()

        bufs = ((dv0, sg0, sem0), (dv1, sg1, sem1))

        def mk(k, bi):
            dv, sg, sem = bufs[bi]
            off = k * CHD
            return (pltpu.make_async_copy(sig_hbm.at[pl.ds(base + off, CHD)],
                                          sg, sem),
                    pltpu.make_async_copy(dst_hbm.at[pl.ds(base + off, CHD)],
                                          dv, sem))

        def fire(k, bi):
            for cp in mk(k, bi):
                cp.start()

        def drain(k, bi):
            for cp in mk(k, bi):
                cp.wait()

        def process(k, bi):
            dv, sg, _ = bufs[bi]
            drain(k, bi)
            pltpu.sync_copy(sg, acc.at[dv], add=True)

        _pipe(nch, fire, process, drain)

        plsc.subcore_barrier()
        pltpu.sync_copy(acc.at[pl.ds(s * ZR, ZR)],
                        ss_hbm.at[c].at[pl.ds(s * ZR, ZR)])

    return k(sig, dst)


def _seg_ssh(sigma, bh, src, dst):
    f32 = jnp.float32
    CHD = 40
    ne = src.shape[0]
    per_tile = ne // NW
    nch = per_tile // CHD

    @functools.partial(
        pl.kernel,
        out_type=jax.ShapeDtypeStruct((NC, NP, D), f32),
        mesh=_sc_mesh(),
        scratch_types=[
            pltpu.VMEM_SHARED((NP, D), f32),
            pltpu.VMEM((per_tile,), jnp.int32),
            pltpu.VMEM((CHD,), jnp.int32),
            pltpu.VMEM((CHD,), jnp.int32),
            pltpu.VMEM((CHD, D), f32),
            pltpu.VMEM((CHD, D), f32),
            pltpu.VMEM((CHD, D), f32),
            pltpu.VMEM((CHD, D), f32),
            pltpu.SemaphoreType.DMA,
            pltpu.SemaphoreType.DMA,
        ],
    )
    def k(sig_hbm, bh_hbm, src_hbm, dst_hbm, ssh_hbm,
          acc, siv, dv0, dv1, sg0, sg1, bh0, bh1, sem0, sem1):
        c = lax.axis_index("c")
        s = lax.axis_index("s")
        base = (c * NS + s) * per_tile
        zvec = jnp.zeros((16,), f32)

        pltpu.sync_copy(src_hbm.at[pl.ds(base, per_tile)], siv)

        @pl.loop(0, CHD)
        def _(r):
            for g in range(D // 16):
                bh0[r, pl.ds(g * 16, 16)] = zvec

        for kk in range(ZR // CHD):
            pltpu.sync_copy(bh0, acc.at[pl.ds(s * ZR + kk * CHD, CHD)])
        plsc.subcore_barrier()

        bufs = ((dv0, sg0, bh0, sem0), (dv1, sg1, bh1, sem1))

        def mk(k, bi):
            dv, sg, bh, sem = bufs[bi]
            off = k * CHD
            return (pltpu.make_async_copy(sig_hbm.at[pl.ds(base + off, CHD)],
                                          sg, sem),
                    pltpu.make_async_copy(dst_hbm.at[pl.ds(base + off, CHD)],
                                          dv, sem),
                    pltpu.make_async_copy(bh_hbm.at[siv.at[pl.ds(off, CHD)]],
                                          bh, sem))

        def fire(k, bi):
            for cp in mk(k, bi):
                cp.start()

        def drain(k, bi):
            for cp in mk(k, bi):
                cp.wait()

        def process(k, bi):
            dv, sg, bh, _ = bufs[bi]
            drain(k, bi)

            @pl.loop(0, CHD)
            def _(r):
                for g in range(D // 16):
                    sl = pl.ds(g * 16, 16)
                    bh[r, sl] = bh[r, sl] * sg[r, sl]

            pltpu.sync_copy(bh, acc.at[dv], add=True)

        _pipe(nch, fire, process, drain)

        plsc.subcore_barrier()
        pltpu.sync_copy(acc.at[pl.ds(s * ZR, ZR)],
                        ssh_hbm.at[c].at[pl.ds(s * ZR, ZR)])

    return k(sigma, bh, src, dst)


def _node_post_body(nf_ref, su_ref, ss1_ref, ssh1_ref, ss2_ref, ssh2_ref,
                    ss3_ref, ssh3_ref, lng_ref, lnb_ref, x_ref):
    ss = (ss1_ref[0] + ss1_ref[1] + ss2_ref[0] + ss2_ref[1]
          + ss3_ref[0] + ss3_ref[1])
    ssh = (ssh1_ref[0] + ssh1_ref[1] + ssh2_ref[0] + ssh2_ref[1]
           + ssh3_ref[0] + ssh3_ref[1])
    h = ssh / (ss + 1e-6)
    t = su_ref[...] + h
    mu = jnp.mean(t, axis=-1, keepdims=True)
    zc = t - mu
    var = jnp.mean(zc * zc, axis=-1, keepdims=True)
    tn = zc * lax.rsqrt(var + 1e-5) * lng_ref[...] + lnb_ref[...]
    x_ref[...] = nf_ref[...] + tn * jax.nn.sigmoid(tn)


def _node_post(nf, su, parts, lng, lnb):
    BN = 2000
    nspec = pl.BlockSpec((BN, D), lambda i: (i, 0))
    pspec = pl.BlockSpec((2, BN, D), lambda i: (0, i, 0))
    wspec = pl.BlockSpec((1, D), lambda i: (0, 0))
    flat = [a for pair in parts for a in pair]
    return pl.pallas_call(
        _node_post_body,
        grid=(N // BN,),
        in_specs=[nspec, nspec] + [pspec] * 6 + [wspec, wspec],
        out_specs=nspec,
        out_shape=jax.ShapeDtypeStruct((N, D), jnp.float32),
    )(nf, su, *flat, lng, lnb)


def kernel(node_feats, edge_feats, edge_index,
           sg_A, sg_B, sg_b, dg_A, dg_B, dg_b, eg_A, eg_B, eg_b,
           du_A, du_B, du_b, su_A, su_B, su_b,
           ln_n_g, ln_n_b, ln_e_g, ln_e_b):
    src = edge_index[0].astype(jnp.int32)
    dst = edge_index[1].astype(jnp.int32)

    r1 = lambda v: v.reshape(1, D)

    es, ed, bh, su = _node_pre(node_feats, sg_A, sg_B, r1(sg_b),
                               dg_A, dg_B, r1(dg_b), du_A, du_B, r1(du_b),
                               su_A, su_B, r1(su_b))

    bounds = [0, 64000, 192000, E]
    srcs = [src[bounds[i]:bounds[i + 1]] for i in range(3)]
    dsts = [dst[bounds[i]:bounds[i + 1]] for i in range(3)]

    gs = [_gather_g(es, ed, srcs[i], dsts[i]) for i in range(3)]

    y = None
    sigs = []
    for i in range(3):
        y, sig = _edge_dense(edge_feats, gs[i], eg_A, eg_B, r1(eg_b),
                             r1(ln_e_g), r1(ln_e_b), y, bounds[i] // BE)
        sigs.append(sig)

    parts = [(_seg_ss(sigs[i], dsts[i]),
              _seg_ssh(sigs[i], bh, srcs[i], dsts[i])) for i in range(3)]

    x = _node_post(node_feats, su, parts, r1(ln_n_g), r1(ln_n_b))
    return (x, y)


# speedup vs baseline: 2.4068x; 1.0484x over previous
import functools

import jax
import jax.numpy as jnp
from jax import lax
from jax.experimental import pallas as pl
from jax.experimental.pallas import tpu as pltpu
from jax.experimental.pallas import tpu_sc as plsc

N = 10000
E = 320000
D = 128
R = 32

NC = 2
NS = 16
NW = NC * NS
CH = 80
NP = 10240
ZR = NP // NS
ZB = 160

BE = 2000


def _pipe(nch, fire, process, drain):
    fire(0, 0)
    fire(1, 1)
    if nch % 2 == 0:
        @pl.loop(0, nch - 2, step=2)
        def _(k):
            process(k, 0)
            fire(k + 2, 0)
            process(k + 1, 1)
            fire(k + 3, 1)

        process(nch - 2, 0)
        process(nch - 1, 1)
    else:
        @pl.loop(0, nch - 2, step=2)
        def _(k):
            process(k, 0)
            fire(k + 2, 0)
            process(k + 1, 1)
            fire(jnp.minimum(k + 3, nch - 1), 1)

        process(nch - 1, 0)
        drain(nch - 1, 1)


def _sc_mesh():
    return plsc.VectorSubcoreMesh(
        core_axis_name="c", subcore_axis_name="s",
        num_cores=NC, num_subcores=NS)


def _node_pre_body(nf_ref, sgA_ref, sgB_ref, sgb_ref, dgA_ref, dgB_ref,
                   dgb_ref, duA_ref, duB_ref, dub_ref,
                   suA_ref, suB_ref, sub_ref,
                   es_ref, ed_ref, bh_ref, su_ref):
    nf = nf_ref[...]
    f32 = jnp.float32

    def lr(A_ref, B_ref, b_ref):
        return jnp.dot(jnp.dot(nf, A_ref[...], preferred_element_type=f32),
                       B_ref[...], preferred_element_type=f32) + b_ref[...]

    es_ref[...] = lr(sgA_ref, sgB_ref, sgb_ref)
    ed_ref[...] = lr(dgA_ref, dgB_ref, dgb_ref)
    bh_ref[...] = lr(duA_ref, duB_ref, dub_ref)
    su_ref[...] = lr(suA_ref, suB_ref, sub_ref)


def _node_pre(nf, sgA, sgB, sgb, dgA, dgB, dgb, duA, duB, dub,
              suA, suB, sub):
    f32 = jnp.float32
    return pl.pallas_call(
        _node_pre_body,
        out_shape=(
            jax.ShapeDtypeStruct((N, D), f32),
            jax.ShapeDtypeStruct((N, D), f32),
            jax.ShapeDtypeStruct((N, D), f32),
            jax.ShapeDtypeStruct((N, D), f32),
        ),
    )(nf, sgA, sgB, sgb, dgA, dgB, dgb, duA, duB, dub, suA, suB, sub)


def _gather_g(es, ed, src, dst):
    f32 = jnp.float32
    ne = src.shape[0]
    per_tile = ne // NW
    nch = per_tile // CH

    @functools.partial(
        pl.kernel,
        out_type=jax.ShapeDtypeStruct((ne, D), f32),
        mesh=_sc_mesh(),
        scratch_types=[
            pltpu.VMEM((per_tile,), jnp.int32),
            pltpu.VMEM((per_tile,), jnp.int32),
            pltpu.VMEM((CH, D), f32),
            pltpu.VMEM((CH, D), f32),
            pltpu.VMEM((CH, D), f32),
            pltpu.VMEM((CH, D), f32),
            pltpu.SemaphoreType.DMA,
            pltpu.SemaphoreType.DMA,
        ],
    )
    def k(es_hbm, ed_hbm, src_hbm, dst_hbm, g_hbm,
          siv, div, gs0, gd0, gs1, gd1, gsem0, gsem1):
        wid = lax.axis_index("c") * NS + lax.axis_index("s")
        base = wid * per_tile
        pltpu.sync_copy(src_hbm.at[pl.ds(base, per_tile)], siv)
        pltpu.sync_copy(dst_hbm.at[pl.ds(base, per_tile)], div)

        bufs = ((gs0, gd0, gsem0), (gs1, gd1, gsem1))

        def mk(k, bi):
            gs, gd, sem = bufs[bi]
            off = k * CH
            return (pltpu.make_async_copy(es_hbm.at[siv.at[pl.ds(off, CH)]],
                                          gs, sem),
                    pltpu.make_async_copy(ed_hbm.at[div.at[pl.ds(off, CH)]],
                                          gd, sem))

        def fire(k, bi):
            for cp in mk(k, bi):
                cp.start()

        def drain(k, bi):
            for cp in mk(k, bi):
                cp.wait()

        def process(k, bi):
            gs, gd, _ = bufs[bi]
            drain(k, bi)

            @pl.loop(0, CH)
            def _(r):
                for gi in range(D // 16):
                    sl = pl.ds(gi * 16, 16)
                    gs[r, sl] = gs[r, sl] + gd[r, sl]

            pltpu.sync_copy(gs, g_hbm.at[pl.ds(base + k * CH, CH)])

        _pipe(nch, fire, process, drain)

    return k(es, ed, src, dst)


def _edge_body(ef_ref, g_ref, egA_ref, egB_ref, egb_ref, lng_ref, lnb_ref,
               *refs, has_ybuf):
    y_ref, sig_ref = refs[-2:]
    f32 = jnp.float32
    ef = ef_ref[...]
    m = (g_ref[...]
         + jnp.dot(jnp.dot(ef, egA_ref[...], preferred_element_type=f32),
                   egB_ref[...], preferred_element_type=f32)
         + egb_ref[...])
    sig_ref[...] = jax.nn.sigmoid(m)
    mu = jnp.mean(m, axis=-1, keepdims=True)
    zc = m - mu
    var = jnp.mean(zc * zc, axis=-1, keepdims=True)
    yn = zc * lax.rsqrt(var + 1e-5) * lng_ref[...] + lnb_ref[...]
    y_ref[...] = ef + yn * jax.nn.sigmoid(yn)


def _edge_dense(ef, g, egA, egB, egb, lng, lnb, y_buf, blk_off):
    f32 = jnp.float32
    ne = g.shape[0]
    nblk = ne // BE
    in_specs = [
        pl.BlockSpec((BE, D), lambda i: (i + blk_off, 0)),
        pl.BlockSpec((BE, D), lambda i: (i, 0)),
        pl.BlockSpec((D, R), lambda i: (0, 0)),
        pl.BlockSpec((R, D), lambda i: (0, 0)),
        pl.BlockSpec((1, D), lambda i: (0, 0)),
        pl.BlockSpec((1, D), lambda i: (0, 0)),
        pl.BlockSpec((1, D), lambda i: (0, 0)),
    ]
    args = [ef, g, egA, egB, egb, lng, lnb]
    aliases = {}
    if y_buf is not None:
        in_specs.append(pl.BlockSpec(memory_space=pl.ANY))
        args.append(y_buf)
        aliases = {7: 0}
    return pl.pallas_call(
        functools.partial(_edge_body, has_ybuf=y_buf is not None),
        grid=(nblk,),
        in_specs=in_specs,
        out_specs=(
            pl.BlockSpec((BE, D), lambda i: (i + blk_off, 0)),
            pl.BlockSpec((BE, D), lambda i: (i, 0)),
        ),
        out_shape=(
            jax.ShapeDtypeStruct((E, D), f32),
            jax.ShapeDtypeStruct((ne, D), f32),
        ),
        input_output_aliases=aliases,
    )(*args)


def _seg_ss(sig, dst):
    f32 = jnp.float32
    CHD = 80
    ne = dst.shape[0]
    per_tile = ne // NW
    nch = per_tile // CHD

    @functools.partial(
        pl.kernel,
        out_type=jax.ShapeDtypeStruct((NC, NP, D), f32),
        mesh=_sc_mesh(),
        scratch_types=[
            pltpu.VMEM_SHARED((NP, D), f32),
            pltpu.VMEM((CHD,), jnp.int32),
            pltpu.VMEM((CHD,), jnp.int32),
            pltpu.VMEM((CHD, D), f32),
            pltpu.VMEM((CHD, D), f32),
            pltpu.SemaphoreType.DMA,
            pltpu.SemaphoreType.DMA,
        ],
    )
    def k(sig_hbm, dst_hbm, ss_hbm, acc, dv0, dv1, sg0, sg1, sem0, sem1):
        c = lax.axis_index("c")
        s = lax.axis_index("s")
        base = (c * NS + s) * per_tile
        zvec = jnp.zeros((16,), f32)

        @pl.loop(0, CHD)
        def _(r):
            for g in range(D // 16):
                sg0[r, pl.ds(g * 16, 16)] = zvec

        for kk in range(ZR // CHD):
            pltpu.sync_copy(sg0, acc.at[pl.ds(s * ZR + kk * CHD, CHD)])
        plsc.subcore_barrier()

        bufs = ((dv0, sg0, sem0), (dv1, sg1, sem1))

        def mk(k, bi):
            dv, sg, sem = bufs[bi]
            off = k * CHD
            return (pltpu.make_async_copy(sig_hbm.at[pl.ds(base + off, CHD)],
                                          sg, sem),
                    pltpu.make_async_copy(dst_hbm.at[pl.ds(base + off, CHD)],
                                          dv, sem))

        def fire(k, bi):
            for cp in mk(k, bi):
                cp.start()

        def drain(k, bi):
            for cp in mk(k, bi):
                cp.wait()

        def process(k, bi):
            dv, sg, _ = bufs[bi]
            drain(k, bi)
            pltpu.sync_copy(sg, acc.at[dv], add=True)

        _pipe(nch, fire, process, drain)

        plsc.subcore_barrier()
        pltpu.sync_copy(acc.at[pl.ds(s * ZR, ZR)],
                        ss_hbm.at[c].at[pl.ds(s * ZR, ZR)])

    return k(sig, dst)


def _seg_ssh(sigma, bh, src, dst):
    f32 = jnp.float32
    CHD = 80
    ne = src.shape[0]
    per_tile = ne // NW
    nch = per_tile // CHD

    @functools.partial(
        pl.kernel,
        out_type=jax.ShapeDtypeStruct((NC, NP, D), f32),
        mesh=_sc_mesh(),
        scratch_types=[
            pltpu.VMEM_SHARED((NP, D), f32),
            pltpu.VMEM((per_tile,), jnp.int32),
            pltpu.VMEM((CHD,), jnp.int32),
            pltpu.VMEM((CHD,), jnp.int32),
            pltpu.VMEM((CHD, D), f32),
            pltpu.VMEM((CHD, D), f32),
            pltpu.VMEM((CHD, D), f32),
            pltpu.VMEM((CHD, D), f32),
            pltpu.SemaphoreType.DMA,
            pltpu.SemaphoreType.DMA,
        ],
    )
    def k(sig_hbm, bh_hbm, src_hbm, dst_hbm, ssh_hbm,
          acc, siv, dv0, dv1, sg0, sg1, bh0, bh1, sem0, sem1):
        c = lax.axis_index("c")
        s = lax.axis_index("s")
        base = (c * NS + s) * per_tile
        zvec = jnp.zeros((16,), f32)

        pltpu.sync_copy(src_hbm.at[pl.ds(base, per_tile)], siv)

        @pl.loop(0, CHD)
        def _(r):
            for g in range(D // 16):
                bh0[r, pl.ds(g * 16, 16)] = zvec

        for kk in range(ZR // CHD):
            pltpu.sync_copy(bh0, acc.at[pl.ds(s * ZR + kk * CHD, CHD)])
        plsc.subcore_barrier()

        bufs = ((dv0, sg0, bh0, sem0), (dv1, sg1, bh1, sem1))

        def mk(k, bi):
            dv, sg, bh, sem = bufs[bi]
            off = k * CHD
            return (pltpu.make_async_copy(sig_hbm.at[pl.ds(base + off, CHD)],
                                          sg, sem),
                    pltpu.make_async_copy(dst_hbm.at[pl.ds(base + off, CHD)],
                                          dv, sem),
                    pltpu.make_async_copy(bh_hbm.at[siv.at[pl.ds(off, CHD)]],
                                          bh, sem))

        def fire(k, bi):
            for cp in mk(k, bi):
                cp.start()

        def drain(k, bi):
            for cp in mk(k, bi):
                cp.wait()

        def process(k, bi):
            dv, sg, bh, _ = bufs[bi]
            drain(k, bi)

            @pl.loop(0, CHD)
            def _(r):
                for g in range(D // 16):
                    sl = pl.ds(g * 16, 16)
                    bh[r, sl] = bh[r, sl] * sg[r, sl]

            pltpu.sync_copy(bh, acc.at[dv], add=True)

        _pipe(nch, fire, process, drain)

        plsc.subcore_barrier()
        pltpu.sync_copy(acc.at[pl.ds(s * ZR, ZR)],
                        ssh_hbm.at[c].at[pl.ds(s * ZR, ZR)])

    return k(sigma, bh, src, dst)


def _node_post_body(nf_ref, su_ref, ss1_ref, ssh1_ref, ss2_ref, ssh2_ref,
                    ss3_ref, ssh3_ref, lng_ref, lnb_ref, x_ref):
    ss = (ss1_ref[0] + ss1_ref[1] + ss2_ref[0] + ss2_ref[1]
          + ss3_ref[0] + ss3_ref[1])
    ssh = (ssh1_ref[0] + ssh1_ref[1] + ssh2_ref[0] + ssh2_ref[1]
           + ssh3_ref[0] + ssh3_ref[1])
    h = ssh / (ss + 1e-6)
    t = su_ref[...] + h
    mu = jnp.mean(t, axis=-1, keepdims=True)
    zc = t - mu
    var = jnp.mean(zc * zc, axis=-1, keepdims=True)
    tn = zc * lax.rsqrt(var + 1e-5) * lng_ref[...] + lnb_ref[...]
    x_ref[...] = nf_ref[...] + tn * jax.nn.sigmoid(tn)


def _node_post(nf, su, parts, lng, lnb):
    BN = 2000
    nspec = pl.BlockSpec((BN, D), lambda i: (i, 0))
    pspec = pl.BlockSpec((2, BN, D), lambda i: (0, i, 0))
    wspec = pl.BlockSpec((1, D), lambda i: (0, 0))
    flat = [a for pair in parts for a in pair]
    return pl.pallas_call(
        _node_post_body,
        grid=(N // BN,),
        in_specs=[nspec, nspec] + [pspec] * 6 + [wspec, wspec],
        out_specs=nspec,
        out_shape=jax.ShapeDtypeStruct((N, D), jnp.float32),
    )(nf, su, *flat, lng, lnb)


def kernel(node_feats, edge_feats, edge_index,
           sg_A, sg_B, sg_b, dg_A, dg_B, dg_b, eg_A, eg_B, eg_b,
           du_A, du_B, du_b, su_A, su_B, su_b,
           ln_n_g, ln_n_b, ln_e_g, ln_e_b):
    src = edge_index[0].astype(jnp.int32)
    dst = edge_index[1].astype(jnp.int32)

    r1 = lambda v: v.reshape(1, D)

    es, ed, bh, su = _node_pre(node_feats, sg_A, sg_B, r1(sg_b),
                               dg_A, dg_B, r1(dg_b), du_A, du_B, r1(du_b),
                               su_A, su_B, r1(su_b))

    bounds = [0, 64000, 192000, E]
    srcs = [src[bounds[i]:bounds[i + 1]] for i in range(3)]
    dsts = [dst[bounds[i]:bounds[i + 1]] for i in range(3)]

    gs = [_gather_g(es, ed, srcs[i], dsts[i]) for i in range(3)]

    y = None
    sigs = []
    for i in range(3):
        y, sig = _edge_dense(edge_feats, gs[i], eg_A, eg_B, r1(eg_b),
                             r1(ln_e_g), r1(ln_e_b), y, bounds[i] // BE)
        sigs.append(sig)

    parts = [(_seg_ss(sigs[i], dsts[i]),
              _seg_ssh(sigs[i], bh, srcs[i], dsts[i])) for i in range(3)]

    x = _node_post(node_feats, su, parts, r1(ln_n_g), r1(ln_n_b))
    return (x, y)
